# Initial kernel scaffold; baseline (speedup 1.0000x reference)
import jax
import jax.numpy as jnp
from jax.experimental import pallas as pl


def kernel(x, y, edge_index, neighbor, edge_weight, W1, b1, W2, b2, Wp1, bp1, gamma, beta_bn, Wp2, bp2):
    raise NotImplementedError("write your pallas kernel here")


# speedup vs baseline: 2.0353x; 2.0353x over previous
import functools

import jax
import jax.numpy as jnp
from jax.experimental import pallas as pl
from jax.experimental.pallas import tpu as pltpu

N = 10000
E = 320000
D = 128
H = 256
R = 128
PH = 512
TEMP = 0.5
BETA = 0.5

_BLK = 1000


def _pred_loss_body(s_ref, wp1_ref, scale_ref, shift_ref, wp2_ref, bp2_ref,
                    out_ref):
    i = pl.program_id(0)

    @pl.when(i == 0)
    def _():
        out_ref[...] = jnp.zeros_like(out_ref)

    s = s_ref[...]
    h = jnp.dot(s, wp1_ref[...], preferred_element_type=jnp.float32)
    h = h * scale_ref[...] + shift_ref[...]
    h = jnp.maximum(h, 0.0)
    p = jnp.dot(h, wp2_ref[...], preferred_element_type=jnp.float32)
    p = jnp.maximum(p + bp2_ref[...], 0.0)

    a = s * (1.0 / TEMP)
    am = jnp.max(a, axis=1, keepdims=True)
    lse_a = jnp.log(jnp.sum(jnp.exp(a - am), axis=1, keepdims=True)) + am
    lpt = a - lse_a
    pt = jnp.exp(lpt)
    b = p * (1.0 / TEMP)
    bm = jnp.max(b, axis=1, keepdims=True)
    lse_b = jnp.log(jnp.sum(jnp.exp(b - bm), axis=1, keepdims=True)) + bm
    lp = b - lse_b
    li = jnp.sum(pt * (lpt - lp), axis=1) * (TEMP * TEMP)
    out_ref[...] += jnp.reshape(jnp.sum(li), (1, 1))


def _pred_loss(student, Wp1, scale, shift, Wp2, bp2):
    grid = (N // _BLK,)
    return pl.pallas_call(
        _pred_loss_body,
        grid=grid,
        in_specs=[
            pl.BlockSpec((_BLK, R), lambda i: (i, 0)),
            pl.BlockSpec((R, 2 * PH), lambda i: (0, 0)),
            pl.BlockSpec((1, 2 * PH), lambda i: (0, 0)),
            pl.BlockSpec((1, 2 * PH), lambda i: (0, 0)),
            pl.BlockSpec((2 * PH, R), lambda i: (0, 0)),
            pl.BlockSpec((1, R), lambda i: (0, 0)),
        ],
        out_specs=pl.BlockSpec((1, 1), lambda i: (0, 0)),
        out_shape=jax.ShapeDtypeStruct((1, 1), jnp.float32),
    )(student, Wp1, scale, shift, Wp2, bp2)


def kernel(x, y, edge_index, neighbor, edge_weight, W1, b1, W2, b2,
           Wp1, bp1, gamma, beta_bn, Wp2, bp2):
    src = edge_index[0]
    dst = edge_index[1]

    deg = jax.ops.segment_sum(edge_weight, dst, num_segments=N)
    dinv = jnp.where(deg > 0, jax.lax.rsqrt(jnp.maximum(deg, 1e-12)), 0.0)
    dcol = dinv[:, None]

    t1 = (x @ W1) * dcol
    agg1 = jnp.zeros((N, H), jnp.float32).at[dst].add(
        edge_weight[:, None] * t1[src])
    h1 = jnp.maximum(agg1 * dcol + b1, 0.0)

    t2 = (h1 @ W2) * dcol
    agg2 = jnp.zeros((N, R), jnp.float32).at[dst].add(
        edge_weight[:, None] * t2[src])
    student = agg2 * dcol + b2

    csum = jnp.sum(student, axis=0)
    S = student.T @ student
    u = csum @ Wp1
    mu = u / N + bp1
    e2 = (jnp.sum(Wp1 * (S @ Wp1), axis=0) + 2.0 * bp1 * u) / N + bp1 * bp1
    var = e2 - mu * mu
    isd = jax.lax.rsqrt(var + 1e-5)
    scale = (isd * gamma)[None, :]
    shift = (beta_bn - mu * isd * gamma)[None, :]

    loss_sum = _pred_loss(student, Wp1, scale, shift, Wp2, bp2[None, :])
    return (student, loss_sum[0, 0] / N)


# speedup vs baseline: 7.7895x; 3.8271x over previous
import functools

import jax
import jax.numpy as jnp
from jax import lax
from jax.experimental import pallas as pl
from jax.experimental.pallas import tpu as pltpu
from jax.experimental.pallas import tpu_sc as plsc

N = 10000
E = 320000
D = 128
H = 256
R = 128
PH = 512
TEMP = 0.5

NC = 2
NS = 16
NW = NC * NS

N2 = 10240
SEG = N2 // NS
EC = 128
NCHUNK = E // EC

_BLK = 1000


def _mesh():
    return plsc.VectorSubcoreMesh(core_axis_name="c", subcore_axis_name="s",
                                  num_cores=NC, num_subcores=NS)


@functools.cache
def _get_deg_kernel():
    @functools.partial(
        pl.kernel,
        out_type=jax.ShapeDtypeStruct((NC, N2), jnp.float32),
        mesh=_mesh(),
        scratch_types=[
            pltpu.VMEM((EC,), jnp.int32),
            pltpu.VMEM((EC,), jnp.float32),
            pltpu.VMEM((SEG,), jnp.float32),
            pltpu.VMEM_SHARED((N2,), jnp.float32),
        ],
    )
    def _deg(dst_hbm, ew_hbm, out_hbm, idx_v, val_v, buf_v, acc_sh):
        c = lax.axis_index("c")
        s = lax.axis_index("s")
        w = c * NS + s

        def zero_body(k, _):
            buf_v[pl.ds(k * 16, 16)] = jnp.zeros((16,), jnp.float32)
            return _

        lax.fori_loop(0, SEG // 16, zero_body, None)
        seg0 = pl.multiple_of(s * SEG, SEG)
        pltpu.sync_copy(buf_v, acc_sh.at[pl.ds(seg0, SEG)])
        plsc.subcore_barrier()

        lo = w * NCHUNK // NW
        hi = (w + 1) * NCHUNK // NW

        def body(t, _):
            base = t * EC
            pltpu.sync_copy(dst_hbm.at[pl.ds(base, EC)], idx_v)
            pltpu.sync_copy(ew_hbm.at[pl.ds(base, EC)], val_v)
            pltpu.sync_copy(val_v, acc_sh.at[idx_v], add=True)
            return _

        lax.fori_loop(lo, hi, body, None)
        plsc.subcore_barrier()
        seg1 = pl.multiple_of(s * SEG, SEG)
        pltpu.sync_copy(acc_sh.at[pl.ds(seg1, SEG)], buf_v)
        pltpu.sync_copy(buf_v, out_hbm.at[c].at[pl.ds(seg1, SEG)])

    return _deg


def _deg_kernel(dst, ew):
    return _get_deg_kernel()(dst, ew)


ROWS_T = N2 // NS
ROWS_C = 128


def _make_agg(split_features: bool, width: int):
    @functools.partial(
        pl.kernel,
        out_type=jax.ShapeDtypeStruct((NC, N2, width), jnp.float32),
        mesh=_mesh(),
        scratch_types=[
            pltpu.VMEM((EC,), jnp.int32),
            pltpu.VMEM((EC,), jnp.int32),
            pltpu.VMEM((EC,), jnp.float32),
            pltpu.VMEM((EC, width), jnp.float32),
            pltpu.VMEM((ROWS_C, width), jnp.float32),
            pltpu.VMEM_SHARED((N2, width), jnp.float32),
        ],
    )
    def _agg(tab_hbm, src_hbm, dst_hbm, ew_hbm, out_hbm,
             sidx_v, didx_v, ew_v, rows_v, zbuf_v, acc_sh):
        c = lax.axis_index("c")
        s = lax.axis_index("s")

        nsl = width // 16

        def zero_body(k, _):
            zbuf_v[k // nsl, pl.ds((k % nsl) * 16, 16)] = (
                jnp.zeros((16,), jnp.float32))
            return _

        lax.fori_loop(0, ROWS_C * width // 16, zero_body, None)
        for k in range(ROWS_T // ROWS_C):
            row = pl.multiple_of(s * ROWS_T + k * ROWS_C, ROWS_C)
            pltpu.sync_copy(zbuf_v, acc_sh.at[pl.ds(row, ROWS_C)])
        plsc.subcore_barrier()

        if split_features:
            lo = s * NCHUNK // NS
            hi = (s + 1) * NCHUNK // NS
        else:
            w = c * NS + s
            lo = w * NCHUNK // NW
            hi = (w + 1) * NCHUNK // NW

        def body(t, _):
            base = t * EC
            pltpu.sync_copy(src_hbm.at[pl.ds(base, EC)], sidx_v)
            pltpu.sync_copy(dst_hbm.at[pl.ds(base, EC)], didx_v)
            pltpu.sync_copy(ew_hbm.at[pl.ds(base, EC)], ew_v)
            if split_features:
                pltpu.sync_copy(tab_hbm.at[c].at[sidx_v], rows_v)
            else:
                pltpu.sync_copy(tab_hbm.at[sidx_v], rows_v)

            def scale_body(g, _):
                wvec = ew_v[pl.ds(g * 16, 16)]
                for l in range(16):
                    wv = wvec[l]
                    i = g * 16 + l
                    for j in range(width // 16):
                        sl = pl.ds(j * 16, 16)
                        rows_v[i, sl] = rows_v[i, sl] * wv
                return _

            lax.fori_loop(0, EC // 16, scale_body, None)
            pltpu.sync_copy(rows_v, acc_sh.at[didx_v], add=True)
            return _

        lax.fori_loop(lo, hi, body, None)
        plsc.subcore_barrier()
        for k in range(ROWS_T // ROWS_C):
            row = pl.multiple_of(s * ROWS_T + k * ROWS_C, ROWS_C)
            pltpu.sync_copy(acc_sh.at[pl.ds(row, ROWS_C)], zbuf_v)
            pltpu.sync_copy(zbuf_v, out_hbm.at[c].at[pl.ds(row, ROWS_C)])

    return _agg


@functools.cache
def _get_agg1_kernel():
    return _make_agg(True, H // 2)


@functools.cache
def _get_agg2_kernel():
    return _make_agg(False, R)


def _agg1_kernel(t1, src, dst, ew):
    return _get_agg1_kernel()(t1, src, dst, ew)


def _agg2_kernel(t2, src, dst, ew):
    return _get_agg2_kernel()(t2, src, dst, ew)


def _mm1_body(degt_ref, x_ref, w1_ref, t1_ref, dinv_ref):
    d = jnp.sum(degt_ref[...], axis=1, keepdims=True)
    dinv = jnp.where(d > 0, lax.rsqrt(jnp.maximum(d, 1e-12)), 0.0)
    dinv_ref[...] = dinv
    hw = jnp.dot(x_ref[...], w1_ref[...], preferred_element_type=jnp.float32)
    t1_ref[0] = hw[:, :H // 2] * dinv
    t1_ref[1] = hw[:, H // 2:] * dinv


def _mm1(degt, x, W1):
    return pl.pallas_call(
        _mm1_body,
        grid=(N // _BLK,),
        in_specs=[
            pl.BlockSpec((_BLK, 2), lambda i: (i, 0)),
            pl.BlockSpec((_BLK, D), lambda i: (i, 0)),
            pl.BlockSpec((D, H), lambda i: (0, 0)),
        ],
        out_specs=[
            pl.BlockSpec((2, _BLK, H // 2), lambda i: (0, i, 0)),
            pl.BlockSpec((_BLK, 1), lambda i: (i, 0)),
        ],
        out_shape=[
            jax.ShapeDtypeStruct((2, N, H // 2), jnp.float32),
            jax.ShapeDtypeStruct((N, 1), jnp.float32),
        ],
    )(degt, x, W1)


def _mm2_body(agg_ref, dinv_ref, b1_ref, w2_ref, t2_ref):
    dinv = dinv_ref[...]
    h0 = jnp.maximum(agg_ref[0] * dinv + b1_ref[0], 0.0)
    h1 = jnp.maximum(agg_ref[1] * dinv + b1_ref[1], 0.0)
    hw = (jnp.dot(h0, w2_ref[0], preferred_element_type=jnp.float32)
          + jnp.dot(h1, w2_ref[1], preferred_element_type=jnp.float32))
    t2_ref[...] = hw * dinv


def _mm2(agg1, dinv, b1r, W2r):
    return pl.pallas_call(
        _mm2_body,
        grid=(N // _BLK,),
        in_specs=[
            pl.BlockSpec((2, _BLK, H // 2), lambda i: (0, i, 0)),
            pl.BlockSpec((_BLK, 1), lambda i: (i, 0)),
            pl.BlockSpec((2, 1, H // 2), lambda i: (0, 0, 0)),
            pl.BlockSpec((2, H // 2, R), lambda i: (0, 0, 0)),
        ],
        out_specs=pl.BlockSpec((_BLK, R), lambda i: (i, 0)),
        out_shape=jax.ShapeDtypeStruct((N, R), jnp.float32),
    )(agg1, dinv, b1r, W2r)


def _stats_body(agg_ref, dinv_ref, b2_ref, s_ref, gram_ref, csum_ref):
    i = pl.program_id(0)

    @pl.when(i == 0)
    def _():
        gram_ref[...] = jnp.zeros_like(gram_ref)
        csum_ref[...] = jnp.zeros_like(csum_ref)

    st = (agg_ref[0] + agg_ref[1]) * dinv_ref[...] + b2_ref[...]
    s_ref[...] = st
    gram_ref[...] += lax.dot_general(st, st, (((0,), (0,)), ((), ())),
                                     preferred_element_type=jnp.float32)
    csum_ref[...] += jnp.sum(st, axis=0, keepdims=True)


def _stats(agg2, dinv, b2r):
    return pl.pallas_call(
        _stats_body,
        grid=(N // _BLK,),
        in_specs=[
            pl.BlockSpec((2, _BLK, R), lambda i: (0, i, 0)),
            pl.BlockSpec((_BLK, 1), lambda i: (i, 0)),
            pl.BlockSpec((1, R), lambda i: (0, 0)),
        ],
        out_specs=[
            pl.BlockSpec((_BLK, R), lambda i: (i, 0)),
            pl.BlockSpec((R, R), lambda i: (0, 0)),
            pl.BlockSpec((1, R), lambda i: (0, 0)),
        ],
        out_shape=[
            jax.ShapeDtypeStruct((N, R), jnp.float32),
            jax.ShapeDtypeStruct((R, R), jnp.float32),
            jax.ShapeDtypeStruct((1, R), jnp.float32),
        ],
    )(agg2, dinv, b2r)


def _bnstat_body(gram_ref, csum_ref, wp1_ref, bp1_ref, g_ref, be_ref,
                 scale_ref, shift_ref):
    wp1 = wp1_ref[...]
    u = jnp.dot(csum_ref[...], wp1, preferred_element_type=jnp.float32)
    mu = u * (1.0 / N) + bp1_ref[...]
    t = jnp.dot(gram_ref[...], wp1, preferred_element_type=jnp.float32)
    e2 = (jnp.sum(wp1 * t, axis=0, keepdims=True)
          + 2.0 * bp1_ref[...] * u) * (1.0 / N) + bp1_ref[...] * bp1_ref[...]
    var = e2 - mu * mu
    isd = lax.rsqrt(var + 1e-5)
    scale = isd * g_ref[...]
    scale_ref[...] = scale
    shift_ref[...] = be_ref[...] - mu * scale


def _bnstat(gram, csum, Wp1, bp1r, gammar, betar):
    return pl.pallas_call(
        _bnstat_body,
        in_specs=[pl.BlockSpec(a.shape, lambda: (0,) * a.ndim)
                  for a in (gram, csum, Wp1, bp1r, gammar, betar)],
        out_specs=[pl.BlockSpec((1, 2 * PH), lambda: (0, 0))] * 2,
        out_shape=[jax.ShapeDtypeStruct((1, 2 * PH), jnp.float32)] * 2,
    )(gram, csum, Wp1, bp1r, gammar, betar)


def _pred_loss_body(s_ref, wp1_ref, scale_ref, shift_ref, wp2_ref, bp2_ref,
                    out_ref):
    i = pl.program_id(0)

    @pl.when(i == 0)
    def _():
        out_ref[...] = jnp.zeros_like(out_ref)

    s = s_ref[...]
    h = jnp.dot(s, wp1_ref[...], preferred_element_type=jnp.float32)
    h = h * scale_ref[...] + shift_ref[...]
    h = jnp.maximum(h, 0.0)
    p = jnp.dot(h, wp2_ref[...], preferred_element_type=jnp.float32)
    p = jnp.maximum(p + bp2_ref[...], 0.0)

    a = s * (1.0 / TEMP)
    am = jnp.max(a, axis=1, keepdims=True)
    lse_a = jnp.log(jnp.sum(jnp.exp(a - am), axis=1, keepdims=True)) + am
    lpt = a - lse_a
    pt = jnp.exp(lpt)
    b = p * (1.0 / TEMP)
    bm = jnp.max(b, axis=1, keepdims=True)
    lse_b = jnp.log(jnp.sum(jnp.exp(b - bm), axis=1, keepdims=True)) + bm
    lp = b - lse_b
    li = jnp.sum(pt * (lpt - lp), axis=1) * (TEMP * TEMP)
    out_ref[...] += jnp.reshape(jnp.sum(li), (1, 1))


def _pred_loss(student, Wp1, scale, shift, Wp2, bp2r):
    return pl.pallas_call(
        _pred_loss_body,
        grid=(N // _BLK,),
        in_specs=[
            pl.BlockSpec((_BLK, R), lambda i: (i, 0)),
            pl.BlockSpec((R, 2 * PH), lambda i: (0, 0)),
            pl.BlockSpec((1, 2 * PH), lambda i: (0, 0)),
            pl.BlockSpec((1, 2 * PH), lambda i: (0, 0)),
            pl.BlockSpec((2 * PH, R), lambda i: (0, 0)),
            pl.BlockSpec((1, R), lambda i: (0, 0)),
        ],
        out_specs=pl.BlockSpec((1, 1), lambda i: (0, 0)),
        out_shape=jax.ShapeDtypeStruct((1, 1), jnp.float32),
    )(student, Wp1, scale, shift, Wp2, bp2r)


def kernel(x, y, edge_index, neighbor, edge_weight, W1, b1, W2, b2,
           Wp1, bp1, gamma, beta_bn, Wp2, bp2):
    src = edge_index[0]
    dst = edge_index[1]

    degp = _deg_kernel(dst, edge_weight)
    degt = jnp.transpose(degp)

    t1, dinv = _mm1(degt, x, W1)

    agg1 = _agg1_kernel(t1, src, dst, edge_weight)

    t2 = _mm2(agg1, dinv, b1.reshape(2, 1, H // 2), W2.reshape(2, H // 2, R))

    agg2 = _agg2_kernel(t2, src, dst, edge_weight)

    student, gram, csum = _stats(agg2, dinv, b2[None, :])
    scale, shift = _bnstat(gram, csum, Wp1, bp1[None, :], gamma[None, :],
                           beta_bn[None, :])
    loss_sum = _pred_loss(student, Wp1, scale, shift, Wp2, bp2[None, :])
    return (student, loss_sum[0, 0] / N)
